# baseline (speedup 1.0000x reference)
import functools

import jax
import jax.numpy as jnp
from jax import lax
from jax.experimental import pallas as pl
from jax.experimental.pallas import tpu as pltpu
from jax.experimental.pallas import tpu_sc as plsc

VOCAB = 1000000
EMB = 64
HID = 100
NCLS = 10
BATCH = 16384
SEQ = 50

NC = 2
NS = 16
NW = NC * NS
LANES = 16

BAGS_PER_W = BATCH // NW
BAGS_PER_STEP = 16
STEPS = BAGS_PER_W // BAGS_PER_STEP


def _sc_bag_mean(x_idx, table):
    mesh = plsc.VectorSubcoreMesh(core_axis_name="c", subcore_axis_name="s")

    @functools.partial(
        pl.kernel,
        out_type=jax.ShapeDtypeStruct((BATCH, EMB), jnp.float32),
        mesh=mesh,
        scratch_types=[
            pltpu.VMEM((2, BAGS_PER_STEP, SEQ), jnp.int32),
            pltpu.VMEM((2, BAGS_PER_STEP, SEQ, EMB), jnp.float32),
            pltpu.VMEM((2, BAGS_PER_STEP, EMB), jnp.float32),
            pltpu.SemaphoreType.DMA,
            pltpu.SemaphoreType.DMA,
        ],
        compiler_params=pltpu.CompilerParams(use_tc_tiling_on_sc=False),
    )
    def sc_kernel(x_hbm, table_hbm, out_hbm, idx_v, rows_v, out_v, gsem, osem):
        wid = lax.axis_index("s") * NC + lax.axis_index("c")
        base_bag = wid * BAGS_PER_W

        def step_bag0(step):
            return pl.multiple_of(base_bag + step * BAGS_PER_STEP,
                                  BAGS_PER_STEP)

        def fire(step, b):
            pltpu.sync_copy(x_hbm.at[pl.ds(step_bag0(step), BAGS_PER_STEP)],
                            idx_v.at[b])
            for j in range(BAGS_PER_STEP):
                pltpu.async_copy(table_hbm.at[idx_v.at[b].at[j]],
                                 rows_v.at[b].at[j], gsem)

        def drain_gathers(b):
            for j in range(BAGS_PER_STEP):
                pltpu.make_async_copy(table_hbm.at[idx_v.at[b].at[j]],
                                      rows_v.at[b].at[j], gsem).wait()

        def drain_out(b):
            pltpu.make_async_copy(
                out_v.at[b], out_hbm.at[pl.ds(0, BAGS_PER_STEP)], osem).wait()

        fire(0, 0)

        @pl.loop(0, STEPS // 2)
        def _(s2):
            for b in range(2):
                step = s2 * 2 + b
                drain_gathers(b)

                @pl.when(step + 1 < STEPS)
                def _():
                    fire(step + 1, 1 - b)

                @pl.when(step >= 2)
                def _():
                    drain_out(b)

                for j in range(BAGS_PER_STEP):

                    def body(t, acc):
                        r = t * 10
                        for rr in range(10):
                            acc = tuple(
                                acc[d] + rows_v[b, j, r + rr,
                                                pl.ds(d * LANES, LANES)]
                                for d in range(EMB // LANES)
                            )
                        return acc

                    zero = jnp.zeros((LANES,), jnp.float32)
                    acc = lax.fori_loop(0, SEQ // 10, body,
                                        (zero,) * (EMB // LANES))
                    for d in range(EMB // LANES):
                        out_v[b, j, pl.ds(d * LANES, LANES)] = (
                            acc[d] * (1.0 / SEQ))
                pltpu.async_copy(out_v.at[b],
                                 out_hbm.at[pl.ds(step_bag0(step),
                                                  BAGS_PER_STEP)], osem)

        drain_out(0)
        drain_out(1)

    return sc_kernel(x_idx, table)


def _mlp_body(x_ref, wi_ref, wo_ref, b_ref, o_ref):
    x = x_ref[...]
    h = lax.dot_general(x, wi_ref[...], (((1,), (1,)), ((), ())),
                        preferred_element_type=jnp.float32)
    logits = lax.dot_general(h, wo_ref[...], (((1,), (1,)), ((), ())),
                             preferred_element_type=jnp.float32) + b_ref[...]
    m = jnp.max(logits, axis=1, keepdims=True)
    e = jnp.exp(logits - m)
    o_ref[...] = e / jnp.sum(e, axis=1, keepdims=True)


def _tc_mlp(embs, W_i2h, W_h2o, b_h2o):
    BLK = 2048
    return pl.pallas_call(
        _mlp_body,
        grid=(BATCH // BLK,),
        in_specs=[
            pl.BlockSpec((BLK, EMB), lambda i: (i, 0)),
            pl.BlockSpec((HID, EMB), lambda i: (0, 0)),
            pl.BlockSpec((NCLS, HID), lambda i: (0, 0)),
            pl.BlockSpec((1, NCLS), lambda i: (0, 0)),
        ],
        out_specs=pl.BlockSpec((BLK, NCLS), lambda i: (i, 0)),
        out_shape=jax.ShapeDtypeStruct((BATCH, NCLS), jnp.float32),
    )(embs, W_i2h, W_h2o, b_h2o)


@jax.jit
def kernel(X, table, W_i2h, W_h2o, b_h2o):
    embs = _sc_bag_mean(X, table)
    return _tc_mlp(embs, W_i2h, W_h2o, b_h2o.reshape(1, NCLS))


# speedup vs baseline: 1.0777x; 1.0777x over previous
import functools

import jax
import jax.numpy as jnp
from jax import lax
from jax.experimental import pallas as pl
from jax.experimental.pallas import tpu as pltpu
from jax.experimental.pallas import tpu_sc as plsc

VOCAB = 1000000
EMB = 64
HID = 100
NCLS = 10
BATCH = 16384
SEQ = 50

NC = 2
NS = 16
NW = NC * NS
LANES = 16

CPAD = 16
BLKV = 8192
BAGS_PER_W = BATCH // NW
BAGS_PER_STEP = 16
STEPS = BAGS_PER_W // BAGS_PER_STEP


def _proj_body(tt_ref, wi_ref, wop_ref, o_ref):
    wc = lax.dot_general(wop_ref[...], wi_ref[...], (((1,), (0,)), ((), ())),
                         preferred_element_type=jnp.float32) * (1.0 / SEQ)
    o_ref[...] = lax.dot_general(tt_ref[...], wc, (((0,), (1,)), ((), ())),
                                 preferred_element_type=jnp.float32)


def _tc_project(tableT, W_i2h, W_h2o_pad):
    return pl.pallas_call(
        _proj_body,
        grid=((VOCAB + BLKV - 1) // BLKV,),
        in_specs=[
            pl.BlockSpec((EMB, BLKV), lambda i: (0, i)),
            pl.BlockSpec((HID, EMB), lambda i: (0, 0)),
            pl.BlockSpec((CPAD, HID), lambda i: (0, 0)),
        ],
        out_specs=pl.BlockSpec((BLKV, CPAD), lambda i: (i, 0)),
        out_shape=jax.ShapeDtypeStruct((VOCAB, CPAD), jnp.float32),
    )(tableT, W_i2h, W_h2o_pad)


def _sc_bag_softmax(x_idx, p_tab, b_pad):
    mesh = plsc.VectorSubcoreMesh(core_axis_name="c", subcore_axis_name="s")

    @functools.partial(
        pl.kernel,
        out_type=jax.ShapeDtypeStruct((BATCH, CPAD), jnp.float32),
        mesh=mesh,
        scratch_types=[
            pltpu.VMEM((2, BAGS_PER_STEP, SEQ), jnp.int32),
            pltpu.VMEM((2, BAGS_PER_STEP, SEQ, CPAD), jnp.float32),
            pltpu.VMEM((2, BAGS_PER_STEP, CPAD), jnp.float32),
            pltpu.VMEM((CPAD,), jnp.float32),
            pltpu.SemaphoreType.DMA,
            pltpu.SemaphoreType.DMA,
        ],
        compiler_params=pltpu.CompilerParams(use_tc_tiling_on_sc=False,
                                             needs_layout_passes=False),
    )
    def sc_kernel(x_hbm, p_hbm, b_hbm, out_hbm, idx_v, rows_v, out_v, b_v,
                  gsem, osem):
        wid = lax.axis_index("s") * NC + lax.axis_index("c")
        base_bag = wid * BAGS_PER_W
        pltpu.sync_copy(b_hbm, b_v)
        bias = b_v[...]
        valid = lax.iota(jnp.int32, CPAD) < NCLS

        def step_bag0(step):
            return pl.multiple_of(base_bag + step * BAGS_PER_STEP,
                                  BAGS_PER_STEP)

        def fire(step, b):
            pltpu.sync_copy(x_hbm.at[pl.ds(step_bag0(step), BAGS_PER_STEP)],
                            idx_v.at[b])
            for j in range(BAGS_PER_STEP):
                pltpu.async_copy(p_hbm.at[idx_v.at[b].at[j]],
                                 rows_v.at[b].at[j], gsem)

        def drain_gathers(b):
            for j in range(BAGS_PER_STEP):
                pltpu.make_async_copy(p_hbm.at[idx_v.at[b].at[j]],
                                      rows_v.at[b].at[j], gsem).wait()

        def drain_out(b):
            pltpu.make_async_copy(
                out_v.at[b], out_hbm.at[pl.ds(0, BAGS_PER_STEP)], osem).wait()

        fire(0, 0)

        @pl.loop(0, STEPS // 2)
        def _(s2):
            for b in range(2):
                step = s2 * 2 + b
                drain_gathers(b)

                @pl.when(step + 1 < STEPS)
                def _():
                    fire(step + 1, 1 - b)

                @pl.when(step >= 2)
                def _():
                    drain_out(b)

                for j in range(BAGS_PER_STEP):

                    def body(t, acc):
                        r = t * 10
                        for rr in range(10):
                            acc = acc + rows_v[b, j, r + rr]
                        return acc

                    acc = lax.fori_loop(0, SEQ // 10, body,
                                        jnp.zeros((CPAD,), jnp.float32))
                    z = acc + bias
                    m = lax.reduce_max(z, axes=(0,))
                    e = jnp.where(valid, jnp.exp(z - m), 0.0)
                    s = lax.reduce_sum(e, axes=(0,))
                    out_v[b, j] = e / s
                pltpu.async_copy(out_v.at[b],
                                 out_hbm.at[pl.ds(step_bag0(step),
                                                  BAGS_PER_STEP)], osem)

        drain_out(0)
        drain_out(1)

    return sc_kernel(x_idx, p_tab, b_pad)


@jax.jit
def kernel(X, table, W_i2h, W_h2o, b_h2o):
    W_h2o_pad = jnp.pad(W_h2o, ((0, CPAD - NCLS), (0, 0)))
    b_pad = jnp.pad(b_h2o, (0, CPAD - NCLS))
    p_tab = _tc_project(table.T, W_i2h, W_h2o_pad)
    out16 = _sc_bag_softmax(X, p_tab, b_pad)
    return out16[:, :NCLS]


# speedup vs baseline: 1.8805x; 1.7450x over previous
import functools

import jax
import jax.numpy as jnp
from jax import lax
from jax.experimental import pallas as pl
from jax.experimental.pallas import tpu as pltpu
from jax.experimental.pallas import tpu_sc as plsc

VOCAB = 1000000
EMB = 64
HID = 100
NCLS = 10
BATCH = 16384
SEQ = 50

NC = 2
NS = 16
NW = NC * NS
LANES = 16

CPAD = 16
GROUPS = 8
BLKT = 4096
CHUNK = GROUPS * BLKT
NCHUNK = (VOCAB + CHUNK - 1) // CHUNK
PQ_ROWS = NCHUNK * BLKT
BAGS_PER_W = BATCH // NW
BAGS_PER_STEP = 16
STEPS = BAGS_PER_W // BAGS_PER_STEP


def _proj_body(tt_ref, wi_ref, wop_ref, o_ref):
    wc = lax.dot_general(wop_ref[...], wi_ref[...], (((1,), (0,)), ((), ())),
                         preferred_element_type=jnp.float32) * (1.0 / SEQ)
    pt = lax.dot_general(wc, tt_ref[...], (((1,), (0,)), ((), ())),
                         preferred_element_type=jnp.float32)
    for j in range(GROUPS):
        o_ref[:, j * CPAD:(j + 1) * CPAD] = lax.transpose(
            pt[:, j * BLKT:(j + 1) * BLKT], (1, 0))


def _tc_project(tableT, W_i2h, W_h2o_pad):
    return pl.pallas_call(
        _proj_body,
        grid=(NCHUNK,),
        in_specs=[
            pl.BlockSpec((EMB, CHUNK), lambda i: (0, i)),
            pl.BlockSpec((HID, EMB), lambda i: (0, 0)),
            pl.BlockSpec((CPAD, HID), lambda i: (0, 0)),
        ],
        out_specs=pl.BlockSpec((BLKT, GROUPS * CPAD), lambda i: (i, 0)),
        out_shape=jax.ShapeDtypeStruct((PQ_ROWS, GROUPS * CPAD), jnp.float32),
    )(tableT, W_i2h, W_h2o_pad)


def _sc_bag_softmax(x_idx, p_tab, b_pad):
    mesh = plsc.VectorSubcoreMesh(core_axis_name="c", subcore_axis_name="s")

    @functools.partial(
        pl.kernel,
        out_type=jax.ShapeDtypeStruct((BATCH, CPAD), jnp.float32),
        mesh=mesh,
        scratch_types=[
            pltpu.VMEM((2, BAGS_PER_STEP, SEQ), jnp.int32),
            pltpu.VMEM((2, BAGS_PER_STEP, SEQ, CPAD), jnp.float32),
            pltpu.VMEM((2, BAGS_PER_STEP, CPAD), jnp.float32),
            pltpu.VMEM((CPAD,), jnp.float32),
            pltpu.SemaphoreType.DMA,
            pltpu.SemaphoreType.DMA,
        ],
        compiler_params=pltpu.CompilerParams(use_tc_tiling_on_sc=False,
                                             needs_layout_passes=False),
    )
    def sc_kernel(x_hbm, p_hbm, b_hbm, out_hbm, idx_v, rows_v, out_v, b_v,
                  gsem, osem):
        wid = lax.axis_index("s") * NC + lax.axis_index("c")
        base_bag = wid * BAGS_PER_W
        pltpu.sync_copy(b_hbm, b_v)
        bias = b_v[...]
        valid = lax.iota(jnp.int32, CPAD) < NCLS

        def step_bag0(step):
            return pl.multiple_of(base_bag + step * BAGS_PER_STEP,
                                  BAGS_PER_STEP)

        def fire(step, b):
            pltpu.sync_copy(x_hbm.at[pl.ds(step_bag0(step), BAGS_PER_STEP)],
                            idx_v.at[b])
            for j in range(BAGS_PER_STEP):
                pltpu.async_copy(p_hbm.at[idx_v.at[b].at[j]],
                                 rows_v.at[b].at[j], gsem)

        def drain_gathers(b):
            for j in range(BAGS_PER_STEP):
                pltpu.make_async_copy(p_hbm.at[idx_v.at[b].at[j]],
                                      rows_v.at[b].at[j], gsem).wait()

        def drain_out(b):
            pltpu.make_async_copy(
                out_v.at[b], out_hbm.at[pl.ds(0, BAGS_PER_STEP)], osem).wait()

        fire(0, 0)

        @pl.loop(0, STEPS // 2)
        def _(s2):
            for b in range(2):
                step = s2 * 2 + b
                drain_gathers(b)

                @pl.when(step + 1 < STEPS)
                def _():
                    fire(step + 1, 1 - b)

                @pl.when(step >= 2)
                def _():
                    drain_out(b)

                for j in range(BAGS_PER_STEP):

                    def body(t, acc):
                        r = t * 10
                        for rr in range(10):
                            acc = acc + rows_v[b, j, r + rr]
                        return acc

                    acc = lax.fori_loop(0, SEQ // 10, body,
                                        jnp.zeros((CPAD,), jnp.float32))
                    z = acc + bias
                    m = lax.reduce_max(z, axes=(0,))
                    e = jnp.where(valid, jnp.exp(z - m), 0.0)
                    s = lax.reduce_sum(e, axes=(0,))
                    out_v[b, j] = e / s
                pltpu.async_copy(out_v.at[b],
                                 out_hbm.at[pl.ds(step_bag0(step),
                                                  BAGS_PER_STEP)], osem)

        drain_out(0)
        drain_out(1)

    return sc_kernel(x_idx, p_tab, b_pad)


@jax.jit
def kernel(X, table, W_i2h, W_h2o, b_h2o):
    W_h2o_pad = jnp.pad(W_h2o, ((0, CPAD - NCLS), (0, 0)))
    b_pad = jnp.pad(b_h2o, (0, CPAD - NCLS))
    pq = _tc_project(table.T, W_i2h, W_h2o_pad)
    p_tab = pq.reshape(PQ_ROWS * GROUPS, CPAD)
    x2 = (((X >> 15) * BLKT) + (X & (BLKT - 1))) * GROUPS + ((X >> 12) & 7)
    out16 = _sc_bag_softmax(x2, p_tab, b_pad)
    return out16[:, :NCLS]


# speedup vs baseline: 3.1911x; 1.6969x over previous
import functools

import jax
import jax.numpy as jnp
from jax import lax
from jax.experimental import pallas as pl
from jax.experimental.pallas import tpu as pltpu
from jax.experimental.pallas import tpu_sc as plsc

VOCAB = 1000000
EMB = 64
HID = 100
NCLS = 10
BATCH = 16384
SEQ = 50

NC = 2
NS = 16
NW = NC * NS
LANES = 16

CPAD = 16
GROUPS = 8
BLKT = 4096
CHUNK = GROUPS * BLKT
NCHUNK = (VOCAB + CHUNK - 1) // CHUNK
PQ_ROWS = NCHUNK * BLKT
BAGS_PER_W = BATCH // NW
BAGS_PER_STEP = 16
STEPS = BAGS_PER_W // BAGS_PER_STEP


def _proj_body(tt_ref, wi_ref, wop_ref, o_ref):
    wc = lax.dot_general(wop_ref[...], wi_ref[...], (((1,), (0,)), ((), ())),
                         preferred_element_type=jnp.float32) * (1.0 / SEQ)
    pt = lax.dot_general(wc, tt_ref[...], (((1,), (0,)), ((), ())),
                         preferred_element_type=jnp.float32)
    stack = jnp.concatenate(
        [pt[:, j * BLKT:(j + 1) * BLKT] for j in range(GROUPS)], axis=0)
    o_ref[...] = lax.transpose(stack, (1, 0))


def _tc_project(tableT, W_i2h, W_h2o_pad):
    return pl.pallas_call(
        _proj_body,
        grid=(NCHUNK,),
        in_specs=[
            pl.BlockSpec((EMB, CHUNK), lambda i: (0, i)),
            pl.BlockSpec((HID, EMB), lambda i: (0, 0)),
            pl.BlockSpec((CPAD, HID), lambda i: (0, 0)),
        ],
        out_specs=pl.BlockSpec((BLKT, GROUPS * CPAD), lambda i: (i, 0)),
        out_shape=jax.ShapeDtypeStruct((PQ_ROWS, GROUPS * CPAD), jnp.float32),
        compiler_params=pltpu.CompilerParams(
            dimension_semantics=("parallel",)),
    )(tableT, W_i2h, W_h2o_pad)


def _sc_bag_softmax(x_idx, p_tab, b_pad):
    mesh = plsc.VectorSubcoreMesh(core_axis_name="c", subcore_axis_name="s")

    @functools.partial(
        pl.kernel,
        out_type=jax.ShapeDtypeStruct((BATCH, CPAD), jnp.float32),
        mesh=mesh,
        scratch_types=[
            pltpu.VMEM((2, BAGS_PER_STEP, SEQ), jnp.int32),
            pltpu.VMEM((2, BAGS_PER_STEP, SEQ, CPAD), jnp.float32),
            pltpu.VMEM((2, BAGS_PER_STEP, CPAD), jnp.float32),
            pltpu.VMEM((CPAD,), jnp.float32),
            pltpu.SemaphoreType.DMA,
            pltpu.SemaphoreType.DMA,
        ],
        compiler_params=pltpu.CompilerParams(use_tc_tiling_on_sc=False,
                                             needs_layout_passes=False),
    )
    def sc_kernel(x_hbm, p_hbm, b_hbm, out_hbm, idx_v, rows_v, out_v, b_v,
                  gsem, osem):
        wid = lax.axis_index("s") * NC + lax.axis_index("c")
        base_bag = wid * BAGS_PER_W
        pltpu.sync_copy(b_hbm, b_v)
        bias = b_v[...]
        valid = lax.iota(jnp.int32, CPAD) < NCLS

        def step_bag0(step):
            return pl.multiple_of(base_bag + step * BAGS_PER_STEP,
                                  BAGS_PER_STEP)

        def fire(step, b):
            pltpu.sync_copy(x_hbm.at[pl.ds(step_bag0(step), BAGS_PER_STEP)],
                            idx_v.at[b])
            for j in range(BAGS_PER_STEP):
                pltpu.async_copy(p_hbm.at[idx_v.at[b].at[j]],
                                 rows_v.at[b].at[j], gsem)

        def drain_gathers(b):
            for j in range(BAGS_PER_STEP):
                pltpu.make_async_copy(p_hbm.at[idx_v.at[b].at[j]],
                                      rows_v.at[b].at[j], gsem).wait()

        def drain_out(b):
            pltpu.make_async_copy(
                out_v.at[b], out_hbm.at[pl.ds(0, BAGS_PER_STEP)], osem).wait()

        fire(0, 0)

        @pl.loop(0, STEPS // 2)
        def _(s2):
            for b in range(2):
                step = s2 * 2 + b
                drain_gathers(b)

                @pl.when(step + 1 < STEPS)
                def _():
                    fire(step + 1, 1 - b)

                @pl.when(step >= 2)
                def _():
                    drain_out(b)

                for j in range(BAGS_PER_STEP):

                    def body(t, acc):
                        r = t * 10
                        for rr in range(10):
                            acc = acc + rows_v[b, j, r + rr]
                        return acc

                    acc = lax.fori_loop(0, SEQ // 10, body,
                                        jnp.zeros((CPAD,), jnp.float32))
                    z = acc + bias
                    m = lax.reduce_max(z, axes=(0,))
                    e = jnp.where(valid, jnp.exp(z - m), 0.0)
                    s = lax.reduce_sum(e, axes=(0,))
                    out_v[b, j] = e / s
                pltpu.async_copy(out_v.at[b],
                                 out_hbm.at[pl.ds(step_bag0(step),
                                                  BAGS_PER_STEP)], osem)

        drain_out(0)
        drain_out(1)

    return sc_kernel(x_idx, p_tab, b_pad)


@jax.jit
def kernel(X, table, W_i2h, W_h2o, b_h2o):
    W_h2o_pad = jnp.pad(W_h2o, ((0, CPAD - NCLS), (0, 0)))
    b_pad = jnp.pad(b_h2o, (0, CPAD - NCLS))
    pq = _tc_project(table.T, W_i2h, W_h2o_pad)
    p_tab = pq.reshape(PQ_ROWS * GROUPS, CPAD)
    x2 = (((X >> 15) * BLKT) + (X & (BLKT - 1))) * GROUPS + ((X >> 12) & 7)
    out16 = _sc_bag_softmax(x2, p_tab, b_pad)
    return out16[:, :NCLS]
